# baseline (speedup 1.0000x reference)
import jax
import jax.numpy as jnp
from jax.experimental import pallas as pl

B, N, F = 32, 512, 128
GCN_UNITS = 32
DENSE_UNITS = 512
NSPLIT = 4
CHUNK = N // NSPLIT


def _gcn_pool_kernel(x_ref, a0, a1, a2, a3, wg_ref, bg_ref, out_ref):
    xf = x_ref[0]
    xw = jnp.dot(xf, wg_ref[:, :], preferred_element_type=jnp.float32)
    bg = bg_ref[0, :]
    s = jnp.zeros((GCN_UNITS,), jnp.float32)
    m = jnp.full((GCN_UNITS,), -jnp.inf, jnp.float32)
    for ar in (a0, a1, a2, a3):
        h = jnp.dot(ar[0], xw, preferred_element_type=jnp.float32)
        s = s + jnp.sum(h, axis=0)
        m = jnp.maximum(m, jnp.max(h, axis=0))
    out_ref[0, 0, :] = s * (1.0 / N) + bg
    out_ref[0, 1, :] = m + bg


def _head_kernel(p_ref, w1_ref, b1_ref, w2_ref, b2_ref, out_ref):
    p = p_ref[:, :, :].reshape(B, 2 * GCN_UNITS)
    z = jnp.dot(p, w1_ref[:, :], preferred_element_type=jnp.float32)
    z = jnp.maximum(z + b1_ref[0, :], 0.0)
    out = jnp.dot(z, w2_ref[:, :], preferred_element_type=jnp.float32)
    out_ref[:, :] = out + b2_ref[0, :]


@jax.jit
def kernel(x, a, W_gcn, b_gcn, W1, b1, W2, b2):
    a_spec = lambda k: pl.BlockSpec((1, CHUNK, N), lambda b, k=k: (b, k, 0))
    pooled = pl.pallas_call(
        _gcn_pool_kernel,
        grid=(B,),
        in_specs=[
            pl.BlockSpec((1, N, F), lambda b: (b, 0, 0)),
            a_spec(0), a_spec(1), a_spec(2), a_spec(3),
            pl.BlockSpec((F, GCN_UNITS), lambda b: (0, 0)),
            pl.BlockSpec((1, GCN_UNITS), lambda b: (0, 0)),
        ],
        out_specs=pl.BlockSpec((1, 2, GCN_UNITS), lambda b: (b, 0, 0)),
        out_shape=jax.ShapeDtypeStruct((B, 2, GCN_UNITS), jnp.float32),
    )(x[..., :F], a, a, a, a, W_gcn, b_gcn.reshape(1, GCN_UNITS))

    out = pl.pallas_call(
        _head_kernel,
        grid=(1,),
        in_specs=[
            pl.BlockSpec((B, 2, GCN_UNITS), lambda i: (0, 0, 0)),
            pl.BlockSpec((2 * GCN_UNITS, DENSE_UNITS), lambda i: (0, 0)),
            pl.BlockSpec((1, DENSE_UNITS), lambda i: (0, 0)),
            pl.BlockSpec((DENSE_UNITS, 1), lambda i: (0, 0)),
            pl.BlockSpec((1, 1), lambda i: (0, 0)),
        ],
        out_specs=pl.BlockSpec((B, 1), lambda i: (0, 0)),
        out_shape=jax.ShapeDtypeStruct((B, 1), jnp.float32),
    )(pooled, W1, b1.reshape(1, DENSE_UNITS), W2, b2.reshape(1, 1))
    return out


# speedup vs baseline: 1.4045x; 1.4045x over previous
import jax
import jax.numpy as jnp
from jax.experimental import pallas as pl

B, N, F = 32, 512, 128
GCN_UNITS = 32
DENSE_UNITS = 512
GPB = 4


def _gcn_pool_kernel(x_ref, a_ref, wg_ref, bg_ref, out_ref):
    bg = bg_ref[0, :]
    for g in range(GPB):
        xw = jnp.dot(x_ref[g], wg_ref[:, :], preferred_element_type=jnp.float32)
        h = jnp.dot(a_ref[g], xw, preferred_element_type=jnp.float32)
        out_ref[g, 0, :] = jnp.mean(h, axis=0) + bg
        out_ref[g, 1, :] = jnp.max(h, axis=0) + bg


def _head_kernel(p_ref, w1_ref, b1_ref, w2_ref, b2_ref, out_ref):
    p = p_ref[:, :, :].reshape(B, 2 * GCN_UNITS)
    z = jnp.dot(p, w1_ref[:, :], preferred_element_type=jnp.float32)
    z = jnp.maximum(z + b1_ref[0, :], 0.0)
    out = jnp.dot(z, w2_ref[:, :], preferred_element_type=jnp.float32)
    out_ref[:, :] = out + b2_ref[0, :]


@jax.jit
def kernel(x, a, W_gcn, b_gcn, W1, b1, W2, b2):
    pooled = pl.pallas_call(
        _gcn_pool_kernel,
        grid=(B // GPB,),
        in_specs=[
            pl.BlockSpec((GPB, N, F), lambda b: (b, 0, 0)),
            pl.BlockSpec((GPB, N, N), lambda b: (b, 0, 0)),
            pl.BlockSpec((F, GCN_UNITS), lambda b: (0, 0)),
            pl.BlockSpec((1, GCN_UNITS), lambda b: (0, 0)),
        ],
        out_specs=pl.BlockSpec((GPB, 2, GCN_UNITS), lambda b: (b, 0, 0)),
        out_shape=jax.ShapeDtypeStruct((B, 2, GCN_UNITS), jnp.float32),
    )(x[..., :F], a, W_gcn, b_gcn.reshape(1, GCN_UNITS))

    out = pl.pallas_call(
        _head_kernel,
        grid=(1,),
        in_specs=[
            pl.BlockSpec((B, 2, GCN_UNITS), lambda i: (0, 0, 0)),
            pl.BlockSpec((2 * GCN_UNITS, DENSE_UNITS), lambda i: (0, 0)),
            pl.BlockSpec((1, DENSE_UNITS), lambda i: (0, 0)),
            pl.BlockSpec((DENSE_UNITS, 1), lambda i: (0, 0)),
            pl.BlockSpec((1, 1), lambda i: (0, 0)),
        ],
        out_specs=pl.BlockSpec((B, 1), lambda i: (0, 0)),
        out_shape=jax.ShapeDtypeStruct((B, 1), jnp.float32),
    )(pooled, W1, b1.reshape(1, DENSE_UNITS), W2, b2.reshape(1, 1))
    return out
